# baseline (speedup 1.0000x reference)
import functools

import jax
import jax.numpy as jnp
from jax import lax
from jax.experimental import pallas as pl
from jax.experimental.pallas import tpu as pltpu
from jax.experimental.pallas import tpu_sc as plsc

_NCLS = 13
_NPTS = 65536
_DF = 256

_NC = 2
_NS = 16
_NW = _NC * _NS
_C = 128
_SC_ROWS = 36864
_RPW = _SC_ROWS // _NW
_NCHUNK = _RPW // _C
_NBANK = 4

_CPAD = 16
_R = 2048
_TC_ROWS = _NPTS - _SC_ROWS
_TC_NBLK = _TC_ROWS // _R
_TC_BLK0 = _SC_ROWS // _R


def _sc_body(feat_hbm, lab_hbm, sumf_out, aux_out,
             fbuf0, fbuf1, lbuf0, lbuf1, af0, ax0,
             labsm, bucket, cnt, off, pos,
             sf0, sf1, sl0, sl1):
    c = lax.axis_index("c")
    s = lax.axis_index("s")
    wid = s * _NC + c
    row0 = wid * _RPW
    zeros16 = jnp.zeros((16,), jnp.float32)

    def _zrow(r, carry):
        for k in range(16):
            af0[r, pl.ds(16 * k, 16)] = zeros16
        ax0[r, pl.ds(0, 16)] = zeros16
        ax0[r, pl.ds(16, 16)] = zeros16
        return carry
    lax.fori_loop(0, 16, _zrow, 0)

    def _start(g, fbuf, lbuf, fsem, lsem):
        base = row0 + g * _C
        pltpu.async_copy(feat_hbm.at[pl.ds(base, _C)], fbuf, fsem)
        pltpu.async_copy(lab_hbm.at[pl.ds(base, _C)], lbuf, lsem)

    def _compute(fbuf, lbuf, fsem, lsem):
        pltpu.make_async_copy(lab_hbm.at[pl.ds(0, _C)], lbuf, lsem).wait()

        def _zc(ci, carry):
            cnt[ci] = 0
            return carry
        lax.fori_loop(0, _NCLS, _zc, 0)

        def _ext(g2, carry):
            labs = lbuf[pl.ds(16 * g2, 16)]
            for j in range(16):
                l = labs[j]
                labsm[16 * g2 + j] = l
                cnt[l] = cnt[l] + 1
            return carry
        lax.fori_loop(0, _C // 16, _ext, 0)

        def _offs(ci, acc):
            off[ci] = acc
            pos[ci] = acc
            return acc + cnt[ci]
        lax.fori_loop(0, _NCLS, _offs, 0)

        def _place(r, carry):
            l = labsm[r]
            p = pos[l]
            bucket[p] = r
            pos[l] = p + 1
            return carry
        lax.fori_loop(0, _C, _place, 0)

        pltpu.make_async_copy(feat_hbm.at[pl.ds(0, _C)], fbuf, fsem).wait()

        def _cls(ci, carry):
            n = cnt[ci]
            o = off[ci]

            def _rows(i, accs):
                r = bucket[o + i]
                vs = [fbuf[r, pl.ds(16 * k, 16)] for k in range(16)]
                new = tuple(accs[k] + vs[k] for k in range(16))
                sq = list(accs[16:])
                for k in range(16):
                    sq[k % 4] = sq[k % 4] + vs[k] * vs[k]
                return new + tuple(sq)

            init = tuple(zeros16 for _ in range(20))
            accs = lax.fori_loop(0, n, _rows, init)
            for k in range(16):
                cur = af0[ci, pl.ds(16 * k, 16)]
                af0[ci, pl.ds(16 * k, 16)] = cur + accs[k]
            sq = ((accs[16] + accs[17]) + (accs[18] + accs[19]))
            c0 = ax0[ci, pl.ds(0, 16)]
            ax0[ci, pl.ds(0, 16)] = c0 + sq
            nf = n.astype(jnp.float32)
            c1 = ax0[ci, pl.ds(16, 16)]
            ax0[ci, pl.ds(16, 16)] = c1 + (zeros16 + nf)
            return carry
        lax.fori_loop(0, _NCLS, _cls, 0)

    _start(0, fbuf0, lbuf0, sf0, sl0)
    _start(1, fbuf1, lbuf1, sf1, sl1)

    def _iter(i, carry):
        g = 2 * i
        _compute(fbuf0, lbuf0, sf0, sl0)

        @pl.when(g + 2 < _NCHUNK)
        def _n0():
            _start(g + 2, fbuf0, lbuf0, sf0, sl0)

        _compute(fbuf1, lbuf1, sf1, sl1)

        @pl.when(g + 3 < _NCHUNK)
        def _n1():
            _start(g + 3, fbuf1, lbuf1, sf1, sl1)
        return carry
    lax.fori_loop(0, _NCHUNK // 2, _iter, 0)
    if _NCHUNK % 2:
        _compute(fbuf0, lbuf0, sf0, sl0)

    pltpu.sync_copy(af0, sumf_out.at[wid])
    pltpu.sync_copy(ax0, aux_out.at[wid])


def _sc_call(features, labels):
    mesh = plsc.VectorSubcoreMesh(core_axis_name="c", subcore_axis_name="s")
    f = functools.partial(
        pl.kernel,
        out_type=[
            jax.ShapeDtypeStruct((_NW, 16, _DF), jnp.float32),
            jax.ShapeDtypeStruct((_NW, 16, 32), jnp.float32),
        ],
        mesh=mesh,
        scratch_types=[
            pltpu.VMEM((_C, _DF), jnp.float32),
            pltpu.VMEM((_C, _DF), jnp.float32),
            pltpu.VMEM((_C,), jnp.int32),
            pltpu.VMEM((_C,), jnp.int32),
            pltpu.VMEM((16, _DF), jnp.float32),
            pltpu.VMEM((16, 32), jnp.float32),
            pltpu.SMEM((_C,), jnp.int32),
            pltpu.SMEM((_C,), jnp.int32),
            pltpu.SMEM((16,), jnp.int32),
            pltpu.SMEM((16,), jnp.int32),
            pltpu.SMEM((16,), jnp.int32),
        ] + [pltpu.SemaphoreType.DMA] * 4,
    )(_sc_body)
    return f(features, labels)


def _tc_body(lab_ref, x_ref, sumf_ref, aux_ref, acc_f, acc_a):
    i = pl.program_id(0)
    x = x_ref[...]
    lab = lab_ref[0]
    cls = lax.broadcasted_iota(jnp.int32, (_CPAD, _R), 0)
    oh = (cls == lab).astype(jnp.float32)
    rowsq = jnp.sum(x * x, axis=1, keepdims=True)
    colid = lax.broadcasted_iota(jnp.int32, (_R, 128), 1)
    aux = jnp.where(colid == 0, rowsq,
                    jnp.where(colid == 1, 1.0, 0.0))
    pf = lax.dot(oh, x, precision=lax.Precision.HIGHEST,
                 preferred_element_type=jnp.float32)
    pa = lax.dot(oh, aux, precision=lax.Precision.HIGHEST,
                 preferred_element_type=jnp.float32)

    @pl.when(i == 0)
    def _init():
        acc_f[...] = pf
        acc_a[...] = pa

    @pl.when(i > 0)
    def _accum():
        acc_f[...] += pf
        acc_a[...] += pa

    @pl.when(i == pl.num_programs(0) - 1)
    def _finish():
        sumf_ref[...] = acc_f[...]
        aux_ref[...] = acc_a[...]


def _tc_call(features, labels3):
    return pl.pallas_call(
        _tc_body,
        grid=(_TC_NBLK,),
        in_specs=[
            pl.BlockSpec((1, 1, _R), lambda i: (i + _TC_BLK0, 0, 0)),
            pl.BlockSpec((_R, _DF), lambda i: (i + _TC_BLK0, 0)),
        ],
        out_specs=[
            pl.BlockSpec((_CPAD, _DF), lambda i: (0, 0)),
            pl.BlockSpec((_CPAD, 128), lambda i: (0, 0)),
        ],
        out_shape=[
            jax.ShapeDtypeStruct((_CPAD, _DF), jnp.float32),
            jax.ShapeDtypeStruct((_CPAD, 128), jnp.float32),
        ],
        scratch_shapes=[
            pltpu.VMEM((_CPAD, _DF), jnp.float32),
            pltpu.VMEM((_CPAD, 128), jnp.float32),
        ],
    )(labels3, features)


def _epi_body(sc_sumf_ref, sc_aux_ref, tc_sumf_ref, tc_aux_ref, out_ref):
    sf = tc_sumf_ref[...]
    aux = sc_aux_ref[0]
    for w in range(1, _NW):
        aux = aux + sc_aux_ref[w]
    for w in range(_NW):
        sf = sf + sc_sumf_ref[w]
    taux = tc_aux_ref[...]
    tcol = lax.broadcasted_iota(jnp.int32, (_CPAD, 128), 1)
    col = lax.broadcasted_iota(jnp.int32, (16, 32), 1)
    sumsq = (jnp.sum(jnp.where(col < 16, aux, 0.0), axis=1, keepdims=True)
             + jnp.sum(jnp.where(tcol == 0, taux, 0.0), axis=1, keepdims=True))
    counts = (jnp.sum(jnp.where(col >= 16, aux, 0.0), axis=1, keepdims=True) / 16.0
              + jnp.sum(jnp.where(tcol == 1, taux, 0.0), axis=1, keepdims=True))
    safe = jnp.maximum(counts, 1.0)
    nrm = jnp.sum(sf * sf, axis=1, keepdims=True)
    var = (sumsq - nrm / safe) / safe
    rid = lax.broadcasted_iota(jnp.int32, (16, 1), 0)
    valid = (counts > 1.0) & (rid < _NCLS)
    vc = jnp.sum(jnp.where(valid, 1.0, 0.0), axis=(0, 1), keepdims=True)
    loss = jnp.sum(jnp.where(valid, var, 0.0), axis=(0, 1), keepdims=True)
    loss = jnp.where(vc > 0, loss / jnp.maximum(vc, 1.0), 0.0)
    out_ref[...] = loss


def kernel(features, labels):
    labels3 = labels.reshape(_NPTS // _R, 1, _R)
    sc_sumf, sc_aux = _sc_call(features, labels)
    tc_sumf, tc_aux = _tc_call(features, labels3)
    out = pl.pallas_call(
        _epi_body,
        out_shape=jax.ShapeDtypeStruct((1, 1), jnp.float32),
    )(sc_sumf, sc_aux, tc_sumf, tc_aux)
    return out[0, 0]


# speedup vs baseline: 1.0830x; 1.0830x over previous
import functools

import jax
import jax.numpy as jnp
from jax import lax
from jax.experimental import pallas as pl
from jax.experimental.pallas import tpu as pltpu
from jax.experimental.pallas import tpu_sc as plsc

_NCLS = 13
_NPTS = 65536
_DF = 256

_NC = 2
_NS = 16
_NW = _NC * _NS
_C = 128
_SC_ROWS = 32768
_RPW = _SC_ROWS // _NW
_NCHUNK = _RPW // _C
_NBANK = 4

_CPAD = 16
_R = 2048
_TC_ROWS = _NPTS - _SC_ROWS
_TC_NBLK = _TC_ROWS // _R
_TC_BLK0 = _SC_ROWS // _R


def _sc_body(feat_hbm, lab_hbm, sumf_out, aux_out,
             fbuf0, fbuf1, lbuf0, lbuf1, af0, ax0,
             labsm, bucket, cnt, off, pos,
             sf0, sf1, sl0, sl1):
    c = lax.axis_index("c")
    s = lax.axis_index("s")
    wid = s * _NC + c
    row0 = wid * _RPW
    zeros16 = jnp.zeros((16,), jnp.float32)

    def _zrow(r, carry):
        for k in range(16):
            af0[r, pl.ds(16 * k, 16)] = zeros16
        ax0[r, pl.ds(0, 16)] = zeros16
        ax0[r, pl.ds(16, 16)] = zeros16
        return carry
    lax.fori_loop(0, 16, _zrow, 0)

    def _start(g, fbuf, lbuf, fsem, lsem):
        base = row0 + g * _C
        pltpu.async_copy(feat_hbm.at[pl.ds(base, _C)], fbuf, fsem)
        pltpu.async_copy(lab_hbm.at[pl.ds(base, _C)], lbuf, lsem)

    def _compute(fbuf, lbuf, fsem, lsem):
        pltpu.make_async_copy(lab_hbm.at[pl.ds(0, _C)], lbuf, lsem).wait()

        def _zc(ci, carry):
            cnt[ci] = 0
            return carry
        lax.fori_loop(0, _NCLS, _zc, 0)

        def _ext(g2, carry):
            labs = lbuf[pl.ds(16 * g2, 16)]
            for j in range(16):
                l = labs[j]
                labsm[16 * g2 + j] = l
                cnt[l] = cnt[l] + 1
            return carry
        lax.fori_loop(0, _C // 16, _ext, 0)

        def _offs(ci, acc):
            off[ci] = acc
            pos[ci] = acc
            return acc + cnt[ci]
        lax.fori_loop(0, _NCLS, _offs, 0)

        def _place(r, carry):
            l = labsm[r]
            p = pos[l]
            bucket[p] = r
            pos[l] = p + 1
            return carry
        lax.fori_loop(0, _C, _place, 0)

        pltpu.make_async_copy(feat_hbm.at[pl.ds(0, _C)], fbuf, fsem).wait()

        def _cls(ci, carry):
            n = cnt[ci]
            o = off[ci]

            def _rows(i, accs):
                r = bucket[o + i]
                vs = [fbuf[r, pl.ds(16 * k, 16)] for k in range(16)]
                new = tuple(accs[k] + vs[k] for k in range(16))
                sq = list(accs[16:])
                for k in range(16):
                    sq[k % 4] = sq[k % 4] + vs[k] * vs[k]
                return new + tuple(sq)

            init = tuple(zeros16 for _ in range(20))
            accs = lax.fori_loop(0, n, _rows, init)
            for k in range(16):
                cur = af0[ci, pl.ds(16 * k, 16)]
                af0[ci, pl.ds(16 * k, 16)] = cur + accs[k]
            sq = ((accs[16] + accs[17]) + (accs[18] + accs[19]))
            c0 = ax0[ci, pl.ds(0, 16)]
            ax0[ci, pl.ds(0, 16)] = c0 + sq
            nf = n.astype(jnp.float32)
            c1 = ax0[ci, pl.ds(16, 16)]
            ax0[ci, pl.ds(16, 16)] = c1 + (zeros16 + nf)
            return carry
        lax.fori_loop(0, _NCLS, _cls, 0)

    _start(0, fbuf0, lbuf0, sf0, sl0)
    _start(1, fbuf1, lbuf1, sf1, sl1)

    def _iter(i, carry):
        g = 2 * i
        _compute(fbuf0, lbuf0, sf0, sl0)

        @pl.when(g + 2 < _NCHUNK)
        def _n0():
            _start(g + 2, fbuf0, lbuf0, sf0, sl0)

        _compute(fbuf1, lbuf1, sf1, sl1)

        @pl.when(g + 3 < _NCHUNK)
        def _n1():
            _start(g + 3, fbuf1, lbuf1, sf1, sl1)
        return carry
    lax.fori_loop(0, _NCHUNK // 2, _iter, 0)
    if _NCHUNK % 2:
        _compute(fbuf0, lbuf0, sf0, sl0)

    pltpu.sync_copy(af0, sumf_out.at[wid])
    pltpu.sync_copy(ax0, aux_out.at[wid])


def _sc_call(features, labels):
    mesh = plsc.VectorSubcoreMesh(core_axis_name="c", subcore_axis_name="s")
    f = functools.partial(
        pl.kernel,
        out_type=[
            jax.ShapeDtypeStruct((_NW, 16, _DF), jnp.float32),
            jax.ShapeDtypeStruct((_NW, 16, 32), jnp.float32),
        ],
        mesh=mesh,
        scratch_types=[
            pltpu.VMEM((_C, _DF), jnp.float32),
            pltpu.VMEM((_C, _DF), jnp.float32),
            pltpu.VMEM((_C,), jnp.int32),
            pltpu.VMEM((_C,), jnp.int32),
            pltpu.VMEM((16, _DF), jnp.float32),
            pltpu.VMEM((16, 32), jnp.float32),
            pltpu.SMEM((_C,), jnp.int32),
            pltpu.SMEM((_C,), jnp.int32),
            pltpu.SMEM((16,), jnp.int32),
            pltpu.SMEM((16,), jnp.int32),
            pltpu.SMEM((16,), jnp.int32),
        ] + [pltpu.SemaphoreType.DMA] * 4,
    )(_sc_body)
    return f(features, labels)


def _tc_body(lab_ref, x_ref, sumf_ref, aux_ref, acc_f, acc_a):
    i = pl.program_id(0)
    x = x_ref[...]
    lab = lab_ref[0]
    cls = lax.broadcasted_iota(jnp.int32, (_CPAD, _R), 0)
    oh = (cls == lab).astype(jnp.float32)
    rowsq = jnp.sum(x * x, axis=1, keepdims=True)
    colid = lax.broadcasted_iota(jnp.int32, (_R, 128), 1)
    aux = jnp.where(colid == 0, rowsq,
                    jnp.where(colid == 1, 1.0, 0.0))
    pf = lax.dot(oh, x, precision=lax.Precision.HIGHEST,
                 preferred_element_type=jnp.float32)
    pa = lax.dot(oh, aux, precision=lax.Precision.HIGHEST,
                 preferred_element_type=jnp.float32)

    @pl.when(i == 0)
    def _init():
        acc_f[...] = pf
        acc_a[...] = pa

    @pl.when(i > 0)
    def _accum():
        acc_f[...] += pf
        acc_a[...] += pa

    @pl.when(i == pl.num_programs(0) - 1)
    def _finish():
        sumf_ref[...] = acc_f[...]
        aux_ref[...] = acc_a[...]


def _tc_call(features, labels3):
    return pl.pallas_call(
        _tc_body,
        grid=(_TC_NBLK,),
        in_specs=[
            pl.BlockSpec((1, 1, _R), lambda i: (i + _TC_BLK0, 0, 0)),
            pl.BlockSpec((_R, _DF), lambda i: (i + _TC_BLK0, 0)),
        ],
        out_specs=[
            pl.BlockSpec((_CPAD, _DF), lambda i: (0, 0)),
            pl.BlockSpec((_CPAD, 128), lambda i: (0, 0)),
        ],
        out_shape=[
            jax.ShapeDtypeStruct((_CPAD, _DF), jnp.float32),
            jax.ShapeDtypeStruct((_CPAD, 128), jnp.float32),
        ],
        scratch_shapes=[
            pltpu.VMEM((_CPAD, _DF), jnp.float32),
            pltpu.VMEM((_CPAD, 128), jnp.float32),
        ],
    )(labels3, features)


def _epi_body(sc_sumf_ref, sc_aux_ref, tc_sumf_ref, tc_aux_ref, out_ref):
    sf = tc_sumf_ref[...]
    aux = sc_aux_ref[0]
    for w in range(1, _NW):
        aux = aux + sc_aux_ref[w]
    for w in range(_NW):
        sf = sf + sc_sumf_ref[w]
    taux = tc_aux_ref[...]
    tcol = lax.broadcasted_iota(jnp.int32, (_CPAD, 128), 1)
    col = lax.broadcasted_iota(jnp.int32, (16, 32), 1)
    sumsq = (jnp.sum(jnp.where(col < 16, aux, 0.0), axis=1, keepdims=True)
             + jnp.sum(jnp.where(tcol == 0, taux, 0.0), axis=1, keepdims=True))
    counts = (jnp.sum(jnp.where(col >= 16, aux, 0.0), axis=1, keepdims=True) / 16.0
              + jnp.sum(jnp.where(tcol == 1, taux, 0.0), axis=1, keepdims=True))
    safe = jnp.maximum(counts, 1.0)
    nrm = jnp.sum(sf * sf, axis=1, keepdims=True)
    var = (sumsq - nrm / safe) / safe
    rid = lax.broadcasted_iota(jnp.int32, (16, 1), 0)
    valid = (counts > 1.0) & (rid < _NCLS)
    vc = jnp.sum(jnp.where(valid, 1.0, 0.0), axis=(0, 1), keepdims=True)
    loss = jnp.sum(jnp.where(valid, var, 0.0), axis=(0, 1), keepdims=True)
    loss = jnp.where(vc > 0, loss / jnp.maximum(vc, 1.0), 0.0)
    out_ref[...] = loss


def kernel(features, labels):
    labels3 = labels.reshape(_NPTS // _R, 1, _R)
    sc_sumf, sc_aux = _sc_call(features, labels)
    tc_sumf, tc_aux = _tc_call(features, labels3)
    out = pl.pallas_call(
        _epi_body,
        out_shape=jax.ShapeDtypeStruct((1, 1), jnp.float32),
    )(sc_sumf, sc_aux, tc_sumf, tc_aux)
    return out[0, 0]


# speedup vs baseline: 1.0873x; 1.0039x over previous
import functools

import jax
import jax.numpy as jnp
from jax import lax
from jax.experimental import pallas as pl
from jax.experimental.pallas import tpu as pltpu
from jax.experimental.pallas import tpu_sc as plsc

_NCLS = 13
_NPTS = 65536
_DF = 256

_NC = 2
_NS = 16
_NW = _NC * _NS
_C = 128
_SC_ROWS = 32768
_RPW = _SC_ROWS // _NW
_NCHUNK = _RPW // _C
_NBANK = 4

_CPAD = 16
_R = 2048
_TC_ROWS = _NPTS - _SC_ROWS
_TC_NBLK = _TC_ROWS // _R
_TC_BLK0 = _SC_ROWS // _R


def _sc_body(feat_hbm, lab_hbm, sumf_out, aux_out,
             fbuf, lbuf, af0, ax0,
             labsm, bucket, cnt, off, pos,
             fsem, lsem):
    c = lax.axis_index("c")
    s = lax.axis_index("s")
    wid = s * _NC + c
    row0 = wid * _RPW
    zeros16 = jnp.zeros((16,), jnp.float32)

    def _zrow(r, carry):
        for k in range(16):
            af0[r, pl.ds(16 * k, 16)] = zeros16
        ax0[r, pl.ds(0, 16)] = zeros16
        ax0[r, pl.ds(16, 16)] = zeros16
        return carry
    lax.fori_loop(0, 16, _zrow, 0)

    def _start(g):
        p = g % 2
        base = row0 + g * _C
        pltpu.async_copy(feat_hbm.at[pl.ds(base, _C)], fbuf.at[p], fsem.at[p])
        pltpu.async_copy(lab_hbm.at[pl.ds(base, _C)], lbuf.at[p], lsem.at[p])

    def _compute(p):
        pltpu.make_async_copy(
            lab_hbm.at[pl.ds(0, _C)], lbuf.at[p], lsem.at[p]).wait()

        def _zc(ci, carry):
            cnt[ci] = 0
            return carry
        lax.fori_loop(0, _NCLS, _zc, 0)

        def _ext(g2, carry):
            labs = lbuf[p, pl.ds(16 * g2, 16)]
            for j in range(16):
                l = labs[j]
                labsm[16 * g2 + j] = l
                cnt[l] = cnt[l] + 1
            return carry
        lax.fori_loop(0, _C // 16, _ext, 0)

        def _offs(ci, acc):
            off[ci] = acc
            pos[ci] = acc
            return acc + cnt[ci]
        lax.fori_loop(0, _NCLS, _offs, 0)

        def _place(r, carry):
            l = labsm[r]
            p = pos[l]
            bucket[p] = r
            pos[l] = p + 1
            return carry
        lax.fori_loop(0, _C, _place, 0)

        pltpu.make_async_copy(
            feat_hbm.at[pl.ds(0, _C)], fbuf.at[p], fsem.at[p]).wait()

        def _cls(ci, carry):
            n = cnt[ci]
            o = off[ci]

            def _rows(i, accs):
                r = bucket[o + i]
                vs = [fbuf[p, r, pl.ds(16 * k, 16)] for k in range(16)]
                new = tuple(accs[k] + vs[k] for k in range(16))
                sq = list(accs[16:])
                for k in range(16):
                    sq[k % 4] = sq[k % 4] + vs[k] * vs[k]
                return new + tuple(sq)

            init = tuple(zeros16 for _ in range(20))
            accs = lax.fori_loop(0, n, _rows, init)
            for k in range(16):
                cur = af0[ci, pl.ds(16 * k, 16)]
                af0[ci, pl.ds(16 * k, 16)] = cur + accs[k]
            sq = ((accs[16] + accs[17]) + (accs[18] + accs[19]))
            c0 = ax0[ci, pl.ds(0, 16)]
            ax0[ci, pl.ds(0, 16)] = c0 + sq
            nf = n.astype(jnp.float32)
            c1 = ax0[ci, pl.ds(16, 16)]
            ax0[ci, pl.ds(16, 16)] = c1 + (zeros16 + nf)
            return carry
        lax.fori_loop(0, _NCLS, _cls, 0)

    _start(0)
    _start(1)

    def _iter(g, carry):
        _compute(g % 2)

        @pl.when(g + 2 < _NCHUNK)
        def _n0():
            _start(g + 2)
        return carry
    lax.fori_loop(0, _NCHUNK, _iter, 0)

    pltpu.sync_copy(af0, sumf_out.at[wid])
    pltpu.sync_copy(ax0, aux_out.at[wid])


def _sc_call(features, labels):
    mesh = plsc.VectorSubcoreMesh(core_axis_name="c", subcore_axis_name="s")
    f = functools.partial(
        pl.kernel,
        out_type=[
            jax.ShapeDtypeStruct((_NW, 16, _DF), jnp.float32),
            jax.ShapeDtypeStruct((_NW, 16, 32), jnp.float32),
        ],
        mesh=mesh,
        scratch_types=[
            pltpu.VMEM((2, _C, _DF), jnp.float32),
            pltpu.VMEM((2, _C), jnp.int32),
            pltpu.VMEM((16, _DF), jnp.float32),
            pltpu.VMEM((16, 32), jnp.float32),
            pltpu.SMEM((_C,), jnp.int32),
            pltpu.SMEM((_C,), jnp.int32),
            pltpu.SMEM((16,), jnp.int32),
            pltpu.SMEM((16,), jnp.int32),
            pltpu.SMEM((16,), jnp.int32),
            pltpu.SemaphoreType.DMA((2,)),
            pltpu.SemaphoreType.DMA((2,)),
        ],
    )(_sc_body)
    return f(features, labels)


def _tc_body(lab_ref, x_ref, sumf_ref, aux_ref, acc_f, acc_a):
    i = pl.program_id(0)
    x = x_ref[...]
    lab = lab_ref[0]
    cls = lax.broadcasted_iota(jnp.int32, (_CPAD, _R), 0)
    oh = (cls == lab).astype(jnp.float32)
    rowsq = jnp.sum(x * x, axis=1, keepdims=True)
    colid = lax.broadcasted_iota(jnp.int32, (_R, 128), 1)
    aux = jnp.where(colid == 0, rowsq,
                    jnp.where(colid == 1, 1.0, 0.0))
    pf = lax.dot(oh, x, precision=lax.Precision.HIGHEST,
                 preferred_element_type=jnp.float32)
    pa = lax.dot(oh, aux, precision=lax.Precision.HIGHEST,
                 preferred_element_type=jnp.float32)

    @pl.when(i == 0)
    def _init():
        acc_f[...] = pf
        acc_a[...] = pa

    @pl.when(i > 0)
    def _accum():
        acc_f[...] += pf
        acc_a[...] += pa

    @pl.when(i == pl.num_programs(0) - 1)
    def _finish():
        sumf_ref[...] = acc_f[...]
        aux_ref[...] = acc_a[...]


def _tc_call(features, labels3):
    return pl.pallas_call(
        _tc_body,
        grid=(_TC_NBLK,),
        in_specs=[
            pl.BlockSpec((1, 1, _R), lambda i: (i + _TC_BLK0, 0, 0)),
            pl.BlockSpec((_R, _DF), lambda i: (i + _TC_BLK0, 0)),
        ],
        out_specs=[
            pl.BlockSpec((_CPAD, _DF), lambda i: (0, 0)),
            pl.BlockSpec((_CPAD, 128), lambda i: (0, 0)),
        ],
        out_shape=[
            jax.ShapeDtypeStruct((_CPAD, _DF), jnp.float32),
            jax.ShapeDtypeStruct((_CPAD, 128), jnp.float32),
        ],
        scratch_shapes=[
            pltpu.VMEM((_CPAD, _DF), jnp.float32),
            pltpu.VMEM((_CPAD, 128), jnp.float32),
        ],
    )(labels3, features)


def _epi_body(sc_sumf_ref, sc_aux_ref, tc_sumf_ref, tc_aux_ref, out_ref):
    sf = tc_sumf_ref[...]
    aux = sc_aux_ref[0]
    for w in range(1, _NW):
        aux = aux + sc_aux_ref[w]
    for w in range(_NW):
        sf = sf + sc_sumf_ref[w]
    taux = tc_aux_ref[...]
    tcol = lax.broadcasted_iota(jnp.int32, (_CPAD, 128), 1)
    col = lax.broadcasted_iota(jnp.int32, (16, 32), 1)
    sumsq = (jnp.sum(jnp.where(col < 16, aux, 0.0), axis=1, keepdims=True)
             + jnp.sum(jnp.where(tcol == 0, taux, 0.0), axis=1, keepdims=True))
    counts = (jnp.sum(jnp.where(col >= 16, aux, 0.0), axis=1, keepdims=True) / 16.0
              + jnp.sum(jnp.where(tcol == 1, taux, 0.0), axis=1, keepdims=True))
    safe = jnp.maximum(counts, 1.0)
    nrm = jnp.sum(sf * sf, axis=1, keepdims=True)
    var = (sumsq - nrm / safe) / safe
    rid = lax.broadcasted_iota(jnp.int32, (16, 1), 0)
    valid = (counts > 1.0) & (rid < _NCLS)
    vc = jnp.sum(jnp.where(valid, 1.0, 0.0), axis=(0, 1), keepdims=True)
    loss = jnp.sum(jnp.where(valid, var, 0.0), axis=(0, 1), keepdims=True)
    loss = jnp.where(vc > 0, loss / jnp.maximum(vc, 1.0), 0.0)
    out_ref[...] = loss


def kernel(features, labels):
    labels3 = labels.reshape(_NPTS // _R, 1, _R)
    sc_sumf, sc_aux = _sc_call(features, labels)
    tc_sumf, tc_aux = _tc_call(features, labels3)
    out = pl.pallas_call(
        _epi_body,
        out_shape=jax.ShapeDtypeStruct((1, 1), jnp.float32),
    )(sc_sumf, sc_aux, tc_sumf, tc_aux)
    return out[0, 0]


# speedup vs baseline: 1.0880x; 1.0007x over previous
import functools

import jax
import jax.numpy as jnp
from jax import lax
from jax.experimental import pallas as pl
from jax.experimental.pallas import tpu as pltpu
from jax.experimental.pallas import tpu_sc as plsc

_NCLS = 13
_NPTS = 65536
_DF = 256

_NC = 2
_NS = 16
_NW = _NC * _NS
_C = 128
_SC_ROWS = 32768
_RPW = _SC_ROWS // _NW
_NCHUNK = _RPW // _C

_CPAD = 16
_R = 2048
_TC_ROWS = _NPTS - _SC_ROWS
_TC_NBLK = _TC_ROWS // _R
_TC_BLK0 = _SC_ROWS // _R


def _sc_body(feat_hbm, lab_hbm, sumf_out, aux_out,
             fbuf, lbuf, af0, ax0,
             labsm, bucket, cnt, off, pos,
             fsem, lsem):
    c = lax.axis_index("c")
    s = lax.axis_index("s")
    wid = s * _NC + c
    row0 = wid * _RPW
    zeros16 = jnp.zeros((16,), jnp.float32)

    def _zrow(r, carry):
        for k in range(16):
            af0[r, pl.ds(16 * k, 16)] = zeros16
        ax0[r, pl.ds(0, 16)] = zeros16
        ax0[r, pl.ds(16, 16)] = zeros16
        return carry
    lax.fori_loop(0, 16, _zrow, 0)

    def _start(g):
        p = g % 2
        base = row0 + g * _C
        pltpu.async_copy(feat_hbm.at[pl.ds(base, _C)], fbuf.at[p], fsem.at[p])
        pltpu.async_copy(lab_hbm.at[pl.ds(base, _C)], lbuf.at[p], lsem.at[p])

    def _compute(p):
        pltpu.make_async_copy(
            lab_hbm.at[pl.ds(0, _C)], lbuf.at[p], lsem.at[p]).wait()

        def _zc(ci, carry):
            cnt[ci] = 0
            return carry
        lax.fori_loop(0, _NCLS, _zc, 0)

        def _ext(g2, carry):
            labs = lbuf[p, pl.ds(16 * g2, 16)]
            for j in range(16):
                l = labs[j]
                labsm[16 * g2 + j] = l
                cnt[l] = cnt[l] + 1
            return carry
        lax.fori_loop(0, _C // 16, _ext, 0)

        def _offs(ci, acc):
            off[ci] = acc
            pos[ci] = acc
            return acc + cnt[ci]
        lax.fori_loop(0, _NCLS, _offs, 0)

        def _place(r, carry):
            l = labsm[r]
            p = pos[l]
            bucket[p] = r
            pos[l] = p + 1
            return carry
        lax.fori_loop(0, _C, _place, 0)

        pltpu.make_async_copy(
            feat_hbm.at[pl.ds(0, _C)], fbuf.at[p], fsem.at[p]).wait()

        def _cls(ci, carry):
            n = cnt[ci]
            o = off[ci]

            def _rows(i, accs):
                r = bucket[o + i]
                vs = [fbuf[p, r, pl.ds(16 * k, 16)] for k in range(16)]
                new = tuple(accs[k] + vs[k] for k in range(16))
                sq = list(accs[16:])
                for k in range(16):
                    sq[k % 4] = sq[k % 4] + vs[k] * vs[k]
                return new + tuple(sq)

            init = tuple(zeros16 for _ in range(20))
            accs = lax.fori_loop(0, n, _rows, init)
            for k in range(16):
                cur = af0[ci, pl.ds(16 * k, 16)]
                af0[ci, pl.ds(16 * k, 16)] = cur + accs[k]
            sq = ((accs[16] + accs[17]) + (accs[18] + accs[19]))
            c0 = ax0[ci, pl.ds(0, 16)]
            ax0[ci, pl.ds(0, 16)] = c0 + sq
            nf = n.astype(jnp.float32)
            c1 = ax0[ci, pl.ds(16, 16)]
            ax0[ci, pl.ds(16, 16)] = c1 + (zeros16 + nf)
            return carry
        lax.fori_loop(0, _NCLS, _cls, 0)

    _start(0)
    _start(1)

    def _iter(g, carry):
        _compute(g % 2)

        @pl.when(g + 2 < _NCHUNK)
        def _n0():
            _start(g + 2)
        return carry
    lax.fori_loop(0, _NCHUNK, _iter, 0)

    pltpu.sync_copy(af0, sumf_out.at[wid])
    pltpu.sync_copy(ax0, aux_out.at[wid])


def _sc_call(features, labels):
    mesh = plsc.VectorSubcoreMesh(core_axis_name="c", subcore_axis_name="s")
    f = functools.partial(
        pl.kernel,
        out_type=[
            jax.ShapeDtypeStruct((_NW, 16, _DF), jnp.float32),
            jax.ShapeDtypeStruct((_NW, 16, 32), jnp.float32),
        ],
        mesh=mesh,
        scratch_types=[
            pltpu.VMEM((2, _C, _DF), jnp.float32),
            pltpu.VMEM((2, _C), jnp.int32),
            pltpu.VMEM((16, _DF), jnp.float32),
            pltpu.VMEM((16, 32), jnp.float32),
            pltpu.SMEM((_C,), jnp.int32),
            pltpu.SMEM((_C,), jnp.int32),
            pltpu.SMEM((16,), jnp.int32),
            pltpu.SMEM((16,), jnp.int32),
            pltpu.SMEM((16,), jnp.int32),
            pltpu.SemaphoreType.DMA((2,)),
            pltpu.SemaphoreType.DMA((2,)),
        ],
    )(_sc_body)
    return f(features, labels)


def _tc_body(lab_ref, x_ref, sumf_ref, aux_ref, acc_f, acc_a):
    i = pl.program_id(0)
    x = x_ref[...]
    lab = lab_ref[0]
    cls = lax.broadcasted_iota(jnp.int32, (_CPAD, _R), 0)
    oh = (cls == lab).astype(jnp.float32)
    rowsq = jnp.sum(x * x, axis=1, keepdims=True)
    colid = lax.broadcasted_iota(jnp.int32, (_R, 128), 1)
    aux = jnp.where(colid == 0, rowsq,
                    jnp.where(colid == 1, 1.0, 0.0))
    pf = lax.dot(oh, x, precision=lax.Precision.HIGHEST,
                 preferred_element_type=jnp.float32)
    pa = lax.dot(oh, aux, precision=lax.Precision.HIGHEST,
                 preferred_element_type=jnp.float32)

    @pl.when(i == 0)
    def _init():
        acc_f[...] = pf
        acc_a[...] = pa

    @pl.when(i > 0)
    def _accum():
        acc_f[...] += pf
        acc_a[...] += pa

    @pl.when(i == pl.num_programs(0) - 1)
    def _finish():
        sumf_ref[...] = acc_f[...]
        aux_ref[...] = acc_a[...]


def _tc_call(features, labels3):
    return pl.pallas_call(
        _tc_body,
        grid=(_TC_NBLK,),
        in_specs=[
            pl.BlockSpec((1, 1, _R), lambda i: (i + _TC_BLK0, 0, 0)),
            pl.BlockSpec((_R, _DF), lambda i: (i + _TC_BLK0, 0)),
        ],
        out_specs=[
            pl.BlockSpec((_CPAD, _DF), lambda i: (0, 0)),
            pl.BlockSpec((_CPAD, 128), lambda i: (0, 0)),
        ],
        out_shape=[
            jax.ShapeDtypeStruct((_CPAD, _DF), jnp.float32),
            jax.ShapeDtypeStruct((_CPAD, 128), jnp.float32),
        ],
        scratch_shapes=[
            pltpu.VMEM((_CPAD, _DF), jnp.float32),
            pltpu.VMEM((_CPAD, 128), jnp.float32),
        ],
    )(labels3, features)


def _epi_body(sc_sumf_ref, sc_aux_ref, tc_sumf_ref, tc_aux_ref, out_ref):
    sf = tc_sumf_ref[...]
    aux = sc_aux_ref[0]
    for w in range(1, _NW):
        aux = aux + sc_aux_ref[w]
    for w in range(_NW):
        sf = sf + sc_sumf_ref[w]
    taux = tc_aux_ref[...]
    tcol = lax.broadcasted_iota(jnp.int32, (_CPAD, 128), 1)
    col = lax.broadcasted_iota(jnp.int32, (16, 32), 1)
    sumsq = (jnp.sum(jnp.where(col < 16, aux, 0.0), axis=1, keepdims=True)
             + jnp.sum(jnp.where(tcol == 0, taux, 0.0), axis=1, keepdims=True))
    counts = (jnp.sum(jnp.where(col >= 16, aux, 0.0), axis=1, keepdims=True) / 16.0
              + jnp.sum(jnp.where(tcol == 1, taux, 0.0), axis=1, keepdims=True))
    safe = jnp.maximum(counts, 1.0)
    nrm = jnp.sum(sf * sf, axis=1, keepdims=True)
    var = (sumsq - nrm / safe) / safe
    rid = lax.broadcasted_iota(jnp.int32, (16, 1), 0)
    valid = (counts > 1.0) & (rid < _NCLS)
    vc = jnp.sum(jnp.where(valid, 1.0, 0.0), axis=(0, 1), keepdims=True)
    loss = jnp.sum(jnp.where(valid, var, 0.0), axis=(0, 1), keepdims=True)
    loss = jnp.where(vc > 0, loss / jnp.maximum(vc, 1.0), 0.0)
    out_ref[...] = loss


def kernel(features, labels):
    labels3 = labels.reshape(_NPTS // _R, 1, _R)
    sc_sumf, sc_aux = _sc_call(features, labels)
    tc_sumf, tc_aux = _tc_call(features, labels3)
    out = pl.pallas_call(
        _epi_body,
        out_shape=jax.ShapeDtypeStruct((1, 1), jnp.float32),
    )(sc_sumf, sc_aux, tc_sumf, tc_aux)
    return out[0, 0]
